# initial kernel scaffold (unmeasured)
import jax
import jax.numpy as jnp
from jax import lax
from jax.experimental import pallas as pl
from jax.experimental.pallas import tpu as pltpu

WORLD = 32
B_SH = 2
SQ = 128
SKV = 128
H_SH = 4
DH = 64
DM = 512
HD_SH = H_SH * DH


def kernel(x, Wq, K_ext, V_ext, Wo):
    def body(x_ref, wq_ref, k_ref, v_ref, wo_ref, out_ref,
             wq_bufs, wo_bufs, k_buf, v_buf, ctx_s,
             wq_ssem, wq_rsem, wo_ssem, wo_rsem, kv_sem):
        my = lax.axis_index("i")
        right = lax.rem(my + 1, WORLD)
        left = lax.rem(my + WORLD - 1, WORLD)

        barrier = pltpu.get_barrier_semaphore()
        pl.semaphore_signal(barrier, inc=1, device_id=(left,),
                            device_id_type=pl.DeviceIdType.MESH)
        pl.semaphore_signal(barrier, inc=1, device_id=(right,),
                            device_id_type=pl.DeviceIdType.MESH)
        pl.semaphore_wait(barrier, 2)

        wq_bufs[0, :, :] = wq_ref[...]
        wo_bufs[0, :, :] = wo_ref[...]
        out_ref[...] = jnp.zeros((B_SH, SQ, DM), jnp.float32)

        xv = x_ref[...].reshape(B_SH * SQ, DM)
        b0 = my * B_SH

        qb = lax.broadcasted_iota(jnp.int32, (SQ, SKV), 0) // 64
        kb = lax.broadcasted_iota(jnp.int32, (SQ, SKV), 1) // 64
        mask = (qb == kb) | ((kb % 4) == (qb % 4))

        def step(t, carry):
            slot = lax.rem(t, 2)
            g = lax.rem(my - t + WORLD, WORLD)

            ck = pltpu.make_async_copy(
                k_ref.at[pl.ds(b0, B_SH), :, pl.ds(g * H_SH, H_SH), :],
                k_buf, kv_sem.at[0])
            cv = pltpu.make_async_copy(
                v_ref.at[pl.ds(b0, B_SH), :, pl.ds(g * H_SH, H_SH), :],
                v_buf, kv_sem.at[1])
            ck.start()
            cv.start()
            ck.wait()
            cv.wait()

            wq_t = wq_bufs[slot]
            q = jnp.dot(xv, wq_t, preferred_element_type=jnp.float32)
            for b in range(B_SH):
                for h in range(H_SH):
                    qbh = q[b * SQ:(b + 1) * SQ, h * DH:(h + 1) * DH]
                    kbh = k_buf[b, :, h, :]
                    s = lax.dot_general(
                        qbh, kbh, (((1,), (1,)), ((), ())),
                        preferred_element_type=jnp.float32) * 0.125
                    s = jnp.where(mask, s, -1e9)
                    m = jnp.max(s, axis=-1, keepdims=True)
                    w = jnp.exp(s - m)
                    w = w / jnp.sum(w, axis=-1, keepdims=True)
                    ctx_bh = jnp.dot(w, v_buf[b, :, h, :],
                                     preferred_element_type=jnp.float32)
                    ctx_s[b * SQ:(b + 1) * SQ, h * DH:(h + 1) * DH] = ctx_bh
            partial = jnp.dot(ctx_s[...], wo_bufs[slot],
                              preferred_element_type=jnp.float32)
            out_ref[...] += partial.reshape(B_SH, SQ, DM)

            @pl.when(t < WORLD - 1)
            def _():
                r_wq = pltpu.make_async_remote_copy(
                    src_ref=wq_bufs.at[slot], dst_ref=wq_bufs.at[1 - slot],
                    send_sem=wq_ssem.at[slot], recv_sem=wq_rsem.at[1 - slot],
                    device_id=(right,), device_id_type=pl.DeviceIdType.MESH)
                r_wo = pltpu.make_async_remote_copy(
                    src_ref=wo_bufs.at[slot], dst_ref=wo_bufs.at[1 - slot],
                    send_sem=wo_ssem.at[slot], recv_sem=wo_rsem.at[1 - slot],
                    device_id=(right,), device_id_type=pl.DeviceIdType.MESH)
                r_wq.start()
                r_wo.start()
                r_wq.wait()
                r_wo.wait()

            return carry

        lax.fori_loop(0, WORLD, step, 0)

    return pl.pallas_call(
        body,
        out_shape=jax.ShapeDtypeStruct((B_SH, SQ, DM), jnp.float32),
        in_specs=[
            pl.BlockSpec(memory_space=pltpu.VMEM),
            pl.BlockSpec(memory_space=pltpu.VMEM),
            pl.BlockSpec(memory_space=pltpu.ANY),
            pl.BlockSpec(memory_space=pltpu.ANY),
            pl.BlockSpec(memory_space=pltpu.VMEM),
        ],
        out_specs=pl.BlockSpec(memory_space=pltpu.VMEM),
        scratch_shapes=[
            pltpu.VMEM((2, DM, HD_SH), jnp.float32),
            pltpu.VMEM((2, HD_SH, DM), jnp.float32),
            pltpu.VMEM((B_SH, SKV, H_SH, DH), jnp.float32),
            pltpu.VMEM((B_SH, SKV, H_SH, DH), jnp.float32),
            pltpu.VMEM((B_SH * SQ, HD_SH), jnp.float32),
            pltpu.SemaphoreType.DMA((2,)),
            pltpu.SemaphoreType.DMA((2,)),
            pltpu.SemaphoreType.DMA((2,)),
            pltpu.SemaphoreType.DMA((2,)),
            pltpu.SemaphoreType.DMA((2,)),
        ],
        compiler_params=pltpu.CompilerParams(collective_id=0),
    )(x, Wq, K_ext, V_ext, Wo)


# baseline (device time: 1261397 ns/iter reference)
import jax
import jax.numpy as jnp
from jax import lax
from jax.experimental import pallas as pl
from jax.experimental.pallas import tpu as pltpu

try:
    jax.block_until_ready(jax.live_arrays())
except Exception:
    pass

WORLD = 32
B_SH = 2
SQ = 128
SKV = 128
H_SH = 4
DH = 64
DM = 512
HD_SH = H_SH * DH


def kernel(x, Wq, K_ext, V_ext, Wo):
    def body(x_ref, wq_ref, k_ref, v_ref, wo_ref, out_ref,
             wq_bufs, wo_bufs, k_buf, v_buf, ctx_s,
             wq_ssem, wq_rsem, wo_ssem, wo_rsem, kv_sem):
        my = lax.axis_index("i")
        right = lax.rem(my + 1, WORLD)
        left = lax.rem(my + WORLD - 1, WORLD)

        barrier = pltpu.get_barrier_semaphore()
        pl.semaphore_signal(barrier, inc=1, device_id=(left,),
                            device_id_type=pl.DeviceIdType.MESH)
        pl.semaphore_signal(barrier, inc=1, device_id=(right,),
                            device_id_type=pl.DeviceIdType.MESH)
        pl.semaphore_wait(barrier, 2)

        wq_bufs[0, :, :] = wq_ref[...]
        wo_bufs[0, :, :] = wo_ref[...]
        out_ref[...] = jnp.zeros((B_SH, SQ, DM), jnp.float32)

        xv = x_ref[...].reshape(B_SH * SQ, DM)
        b0 = my * B_SH

        qb = lax.broadcasted_iota(jnp.int32, (SQ, SKV), 0) // 64
        kb = lax.broadcasted_iota(jnp.int32, (SQ, SKV), 1) // 64
        mask = (qb == kb) | ((kb % 4) == (qb % 4))

        def step(t, carry):
            slot = lax.rem(t, 2)
            g = lax.rem(my - t + WORLD, WORLD)

            ck = pltpu.make_async_copy(
                k_ref.at[pl.ds(b0, B_SH), :, pl.ds(g * H_SH, H_SH), :],
                k_buf, kv_sem.at[0])
            cv = pltpu.make_async_copy(
                v_ref.at[pl.ds(b0, B_SH), :, pl.ds(g * H_SH, H_SH), :],
                v_buf, kv_sem.at[1])
            ck.start()
            cv.start()
            ck.wait()
            cv.wait()

            wq_t = wq_bufs[slot]
            q = jnp.dot(xv, wq_t, preferred_element_type=jnp.float32)
            for b in range(B_SH):
                for h in range(H_SH):
                    qbh = q[b * SQ:(b + 1) * SQ, h * DH:(h + 1) * DH]
                    kbh = k_buf[b, :, h, :]
                    s = lax.dot_general(
                        qbh, kbh, (((1,), (1,)), ((), ())),
                        preferred_element_type=jnp.float32) * 0.125
                    s = jnp.where(mask, s, -1e9)
                    m = jnp.max(s, axis=-1, keepdims=True)
                    w = jnp.exp(s - m)
                    w = w / jnp.sum(w, axis=-1, keepdims=True)
                    ctx_bh = jnp.dot(w, v_buf[b, :, h, :],
                                     preferred_element_type=jnp.float32)
                    ctx_s[b * SQ:(b + 1) * SQ, h * DH:(h + 1) * DH] = ctx_bh
            partial = jnp.dot(ctx_s[...], wo_bufs[slot],
                              preferred_element_type=jnp.float32)
            out_ref[...] += partial.reshape(B_SH, SQ, DM)

            @pl.when(t < WORLD - 1)
            def _():
                r_wq = pltpu.make_async_remote_copy(
                    src_ref=wq_bufs.at[slot], dst_ref=wq_bufs.at[1 - slot],
                    send_sem=wq_ssem.at[slot], recv_sem=wq_rsem.at[1 - slot],
                    device_id=(right,), device_id_type=pl.DeviceIdType.MESH)
                r_wo = pltpu.make_async_remote_copy(
                    src_ref=wo_bufs.at[slot], dst_ref=wo_bufs.at[1 - slot],
                    send_sem=wo_ssem.at[slot], recv_sem=wo_rsem.at[1 - slot],
                    device_id=(right,), device_id_type=pl.DeviceIdType.MESH)
                r_wq.start()
                r_wo.start()
                r_wq.wait()
                r_wo.wait()

            return carry

        lax.fori_loop(0, WORLD, step, 0)

    return pl.pallas_call(
        body,
        out_shape=jax.ShapeDtypeStruct((B_SH, SQ, DM), jnp.float32),
        in_specs=[
            pl.BlockSpec(memory_space=pltpu.VMEM),
            pl.BlockSpec(memory_space=pltpu.VMEM),
            pl.BlockSpec(memory_space=pl.ANY),
            pl.BlockSpec(memory_space=pl.ANY),
            pl.BlockSpec(memory_space=pltpu.VMEM),
        ],
        out_specs=pl.BlockSpec(memory_space=pltpu.VMEM),
        scratch_shapes=[
            pltpu.VMEM((2, DM, HD_SH), jnp.float32),
            pltpu.VMEM((2, HD_SH, DM), jnp.float32),
            pltpu.VMEM((B_SH, SKV, H_SH, DH), jnp.float32),
            pltpu.VMEM((B_SH, SKV, H_SH, DH), jnp.float32),
            pltpu.VMEM((B_SH * SQ, HD_SH), jnp.float32),
            pltpu.SemaphoreType.DMA((2,)),
            pltpu.SemaphoreType.DMA((2,)),
            pltpu.SemaphoreType.DMA((2,)),
            pltpu.SemaphoreType.DMA((2,)),
            pltpu.SemaphoreType.DMA((2,)),
        ],
        compiler_params=pltpu.CompilerParams(collective_id=0),
    )(x, Wq, K_ext, V_ext, Wo)


# device time: 966071 ns/iter; 1.3057x vs baseline; 1.3057x over previous
import jax
import jax.numpy as jnp
from jax import lax
from jax.experimental import pallas as pl
from jax.experimental.pallas import tpu as pltpu

try:
    jax.block_until_ready(jax.live_arrays())
except Exception:
    pass

WORLD = 32
B_SH = 2
SQ = 128
SKV = 128
H_SH = 4
DH = 64
DM = 512
HD_SH = H_SH * DH


def kernel(x, Wq, K_ext, V_ext, Wo):
    def body(x_ref, wq_ref, k_ref, v_ref, wo_ref, out_ref,
             wq_bufs, wo_bufs, k_bufs, v_bufs, ctx_s,
             wq_ssem, wq_rsem, wo_ssem, wo_rsem, k_sems, v_sems):
        my = lax.axis_index("i")
        right = lax.rem(my + 1, WORLD)
        left = lax.rem(my + WORLD - 1, WORLD)
        b0 = my * B_SH

        def kv_copies(t):
            slot = lax.rem(t, 2)
            g = lax.rem(my - t + WORLD, WORLD)
            cps = []
            for h in range(H_SH):
                cps.append(pltpu.make_async_copy(
                    k_ref.at[pl.ds(b0, B_SH), :, g * H_SH + h, :],
                    k_bufs.at[slot * H_SH + h], k_sems.at[slot, h]))
                cps.append(pltpu.make_async_copy(
                    v_ref.at[pl.ds(b0, B_SH), :, g * H_SH + h, :],
                    v_bufs.at[slot * H_SH + h], v_sems.at[slot, h]))
            return cps

        def weight_rdmas(t):
            r_wq = pltpu.make_async_remote_copy(
                src_ref=wq_bufs.at[t], dst_ref=wq_bufs.at[t + 1],
                send_sem=wq_ssem.at[t], recv_sem=wq_rsem.at[t + 1],
                device_id=(right,), device_id_type=pl.DeviceIdType.MESH)
            r_wo = pltpu.make_async_remote_copy(
                src_ref=wo_bufs.at[t], dst_ref=wo_bufs.at[t + 1],
                send_sem=wo_ssem.at[t], recv_sem=wo_rsem.at[t + 1],
                device_id=(right,), device_id_type=pl.DeviceIdType.MESH)
            return r_wq, r_wo

        barrier = pltpu.get_barrier_semaphore()
        pl.semaphore_signal(barrier, inc=1, device_id=(left,),
                            device_id_type=pl.DeviceIdType.MESH)
        pl.semaphore_signal(barrier, inc=1, device_id=(right,),
                            device_id_type=pl.DeviceIdType.MESH)
        pl.semaphore_wait(barrier, 2)

        wq_bufs[0, :, :] = wq_ref[...].astype(jnp.bfloat16)
        wo_bufs[0, :, :] = wo_ref[...].astype(jnp.bfloat16)
        out_ref[...] = jnp.zeros((B_SH, SQ, DM), jnp.float32)

        xv = x_ref[...].reshape(B_SH * SQ, DM).astype(jnp.bfloat16)

        qb = lax.broadcasted_iota(jnp.int32, (SQ, SKV), 0) // 64
        kb = lax.broadcasted_iota(jnp.int32, (SQ, SKV), 1) // 64
        mask = (qb == kb) | ((kb % 4) == (qb % 4))

        for cp in kv_copies(0):
            cp.start()

        def step(t, carry):
            slot = lax.rem(t, 2)

            @pl.when(t < WORLD - 1)
            def _():
                r_wq, r_wo = weight_rdmas(t)
                r_wq.start()
                r_wo.start()
                for cp in kv_copies(t + 1):
                    cp.start()

            for cp in kv_copies(t):
                cp.wait()

            q = jnp.dot(xv, wq_bufs[t],
                        preferred_element_type=jnp.float32)
            q = q.astype(jnp.bfloat16)
            for b in range(B_SH):
                for h in range(H_SH):
                    qbh = q[b * SQ:(b + 1) * SQ, h * DH:(h + 1) * DH]
                    kbh = k_bufs[slot * H_SH + h, b].astype(jnp.bfloat16)
                    s = lax.dot_general(
                        qbh, kbh, (((1,), (1,)), ((), ())),
                        preferred_element_type=jnp.float32) * 0.125
                    s = jnp.where(mask, s, -1e9)
                    m = jnp.max(s, axis=-1, keepdims=True)
                    w = jnp.exp(s - m)
                    w = (w / jnp.sum(w, axis=-1, keepdims=True))
                    ctx_bh = jnp.dot(
                        w.astype(jnp.bfloat16),
                        v_bufs[slot * H_SH + h, b].astype(jnp.bfloat16),
                        preferred_element_type=jnp.float32)
                    ctx_s[b * SQ:(b + 1) * SQ, h * DH:(h + 1) * DH] = (
                        ctx_bh.astype(jnp.bfloat16))
            partial = jnp.dot(ctx_s[...], wo_bufs[t],
                              preferred_element_type=jnp.float32)
            out_ref[...] += partial.reshape(B_SH, SQ, DM)

            @pl.when(t < WORLD - 1)
            def _():
                r_wq, r_wo = weight_rdmas(t)
                r_wq.wait()
                r_wo.wait()

            return carry

        lax.fori_loop(0, WORLD, step, 0)

    return pl.pallas_call(
        body,
        out_shape=jax.ShapeDtypeStruct((B_SH, SQ, DM), jnp.float32),
        in_specs=[
            pl.BlockSpec(memory_space=pltpu.VMEM),
            pl.BlockSpec(memory_space=pltpu.VMEM),
            pl.BlockSpec(memory_space=pl.ANY),
            pl.BlockSpec(memory_space=pl.ANY),
            pl.BlockSpec(memory_space=pltpu.VMEM),
        ],
        out_specs=pl.BlockSpec(memory_space=pltpu.VMEM),
        scratch_shapes=[
            pltpu.VMEM((WORLD, DM, HD_SH), jnp.bfloat16),
            pltpu.VMEM((WORLD, HD_SH, DM), jnp.bfloat16),
            pltpu.VMEM((2 * H_SH, B_SH, SKV, DH), jnp.float32),
            pltpu.VMEM((2 * H_SH, B_SH, SKV, DH), jnp.float32),
            pltpu.VMEM((B_SH * SQ, HD_SH), jnp.bfloat16),
            pltpu.SemaphoreType.DMA((WORLD,)),
            pltpu.SemaphoreType.DMA((WORLD,)),
            pltpu.SemaphoreType.DMA((WORLD,)),
            pltpu.SemaphoreType.DMA((WORLD,)),
            pltpu.SemaphoreType.DMA((2, H_SH)),
            pltpu.SemaphoreType.DMA((2, H_SH)),
        ],
        compiler_params=pltpu.CompilerParams(collective_id=0),
    )(x, Wq, K_ext, V_ext, Wo)


# device time: 951910 ns/iter; 1.3251x vs baseline; 1.0149x over previous
import jax
import jax.numpy as jnp
from jax import lax
from jax.experimental import pallas as pl
from jax.experimental.pallas import tpu as pltpu

try:
    jax.block_until_ready(jax.live_arrays())
except Exception:
    pass

WORLD = 32
B_SH = 2
SQ = 128
SKV = 128
H_SH = 4
DH = 64
DM = 512
HD_SH = H_SH * DH


def kernel(x, Wq, K_ext, V_ext, Wo):
    def body(x_ref, wq_ref, k_ref, v_ref, wo_ref, out_ref,
             wq_bufs, wo_bufs, k_bufs, v_bufs, ctx_s,
             wq_ssem, wq_rsem, wo_ssem, wo_rsem, k_sems, v_sems):
        my = lax.axis_index("i")
        right = lax.rem(my + 1, WORLD)
        left = lax.rem(my + WORLD - 1, WORLD)
        b0 = my * B_SH

        def kv_copies(t):
            slot = lax.rem(t, 2)
            g = lax.rem(my - t + WORLD, WORLD)
            cps = []
            for h in range(H_SH):
                cps.append(pltpu.make_async_copy(
                    k_ref.at[pl.ds(b0, B_SH), :, g * H_SH + h, :],
                    k_bufs.at[slot * H_SH + h], k_sems.at[slot, h]))
                cps.append(pltpu.make_async_copy(
                    v_ref.at[pl.ds(b0, B_SH), :, g * H_SH + h, :],
                    v_bufs.at[slot * H_SH + h], v_sems.at[slot, h]))
            return cps

        def weight_rdmas(t):
            r_wq = pltpu.make_async_remote_copy(
                src_ref=wq_bufs.at[t], dst_ref=wq_bufs.at[t + 1],
                send_sem=wq_ssem.at[t], recv_sem=wq_rsem.at[t + 1],
                device_id=(right,), device_id_type=pl.DeviceIdType.MESH)
            r_wo = pltpu.make_async_remote_copy(
                src_ref=wo_bufs.at[t], dst_ref=wo_bufs.at[t + 1],
                send_sem=wo_ssem.at[t], recv_sem=wo_rsem.at[t + 1],
                device_id=(right,), device_id_type=pl.DeviceIdType.MESH)
            return r_wq, r_wo

        barrier = pltpu.get_barrier_semaphore()
        pl.semaphore_signal(barrier, inc=1, device_id=(left,),
                            device_id_type=pl.DeviceIdType.MESH)
        pl.semaphore_signal(barrier, inc=1, device_id=(right,),
                            device_id_type=pl.DeviceIdType.MESH)
        pl.semaphore_wait(barrier, 2)

        wq_bufs[0, :, :] = wq_ref[...].astype(jnp.bfloat16)
        wo_bufs[0, :, :] = wo_ref[...].astype(jnp.bfloat16)
        out_ref[...] = jnp.zeros((B_SH, SQ, DM), jnp.float32)

        xv = x_ref[...].reshape(B_SH * SQ, DM).astype(jnp.bfloat16)

        rblk = lax.broadcasted_iota(jnp.int32, (B_SH * SQ, B_SH * SKV), 0) // 64
        cblk = lax.broadcasted_iota(jnp.int32, (B_SH * SQ, B_SH * SKV), 1) // 64
        bias = jnp.where(rblk == cblk, 0.0, -1e9).astype(jnp.float32)

        for cp in kv_copies(0):
            cp.start()

        def step(t, carry):
            slot = lax.rem(t, 2)

            @pl.when(t < WORLD - 1)
            def _():
                r_wq, r_wo = weight_rdmas(t)
                r_wq.start()
                r_wo.start()
                for cp in kv_copies(t + 1):
                    cp.start()

            for cp in kv_copies(t):
                cp.wait()

            q = jnp.dot(xv, wq_bufs[t],
                        preferred_element_type=jnp.float32)
            q = q.astype(jnp.bfloat16)
            for h in range(H_SH):
                qh = q[:, h * DH:(h + 1) * DH]
                kh = k_bufs[slot * H_SH + h].reshape(
                    B_SH * SKV, DH).astype(jnp.bfloat16)
                s = lax.dot_general(
                    qh, kh, (((1,), (1,)), ((), ())),
                    preferred_element_type=jnp.float32) * 0.125 + bias
                w = jnp.exp(s)
                w = w / jnp.sum(w, axis=-1, keepdims=True)
                vh = v_bufs[slot * H_SH + h].reshape(
                    B_SH * SKV, DH).astype(jnp.bfloat16)
                ctx_h = jnp.dot(w.astype(jnp.bfloat16), vh,
                                preferred_element_type=jnp.float32)
                ctx_s[:, h * DH:(h + 1) * DH] = ctx_h.astype(jnp.bfloat16)
            partial = jnp.dot(ctx_s[...], wo_bufs[t],
                              preferred_element_type=jnp.float32)
            out_ref[...] += partial.reshape(B_SH, SQ, DM)

            @pl.when(t < WORLD - 1)
            def _():
                r_wq, r_wo = weight_rdmas(t)
                r_wq.wait()
                r_wo.wait()

            return carry

        lax.fori_loop(0, WORLD, step, 0)

    return pl.pallas_call(
        body,
        out_shape=jax.ShapeDtypeStruct((B_SH, SQ, DM), jnp.float32),
        in_specs=[
            pl.BlockSpec(memory_space=pltpu.VMEM),
            pl.BlockSpec(memory_space=pltpu.VMEM),
            pl.BlockSpec(memory_space=pl.ANY),
            pl.BlockSpec(memory_space=pl.ANY),
            pl.BlockSpec(memory_space=pltpu.VMEM),
        ],
        out_specs=pl.BlockSpec(memory_space=pltpu.VMEM),
        scratch_shapes=[
            pltpu.VMEM((WORLD, DM, HD_SH), jnp.bfloat16),
            pltpu.VMEM((WORLD, HD_SH, DM), jnp.bfloat16),
            pltpu.VMEM((2 * H_SH, B_SH, SKV, DH), jnp.float32),
            pltpu.VMEM((2 * H_SH, B_SH, SKV, DH), jnp.float32),
            pltpu.VMEM((B_SH * SQ, HD_SH), jnp.bfloat16),
            pltpu.SemaphoreType.DMA((WORLD,)),
            pltpu.SemaphoreType.DMA((WORLD,)),
            pltpu.SemaphoreType.DMA((WORLD,)),
            pltpu.SemaphoreType.DMA((WORLD,)),
            pltpu.SemaphoreType.DMA((2, H_SH)),
            pltpu.SemaphoreType.DMA((2, H_SH)),
        ],
        compiler_params=pltpu.CompilerParams(collective_id=0),
    )(x, Wq, K_ext, V_ext, Wo)
